# Initial kernel scaffold; baseline (speedup 1.0000x reference)
import jax
import jax.numpy as jnp
from jax.experimental import pallas as pl


def kernel(acts, table):
    raise NotImplementedError("write your pallas kernel here")


# speedup vs baseline: 1.5096x; 1.5096x over previous
import functools

import jax
import jax.numpy as jnp
from jax import lax
from jax.experimental import pallas as pl
from jax.experimental.pallas import tpu as pltpu
from jax.experimental.pallas import tpu_sc as plsc

ACT_DIM = 18
D_EMBED = 64

NUM_CORES = 2
NUM_SUBCORES = 16
NW = NUM_CORES * NUM_SUBCORES

CH = 512


def _tanh16(v):
  return 1.0 - 2.0 / (jnp.exp(2.0 * v) + 1.0)


@functools.partial(jax.jit, static_argnames=("n_chunks",))
def _sc_embed(acts_flat, table_flat, n_chunks):
  n = acts_flat.shape[0]

  mesh = plsc.VectorSubcoreMesh(
      core_axis_name="c", subcore_axis_name="s",
      num_cores=NUM_CORES, num_subcores=NUM_SUBCORES)

  @functools.partial(
      pl.kernel,
      out_type=jax.ShapeDtypeStruct((n, D_EMBED), jnp.float32),
      mesh=mesh,
      compiler_params=pltpu.CompilerParams(needs_layout_passes=False),
      scratch_types=[
          pltpu.VMEM((ACT_DIM * D_EMBED,), jnp.float32),
          pltpu.VMEM((CH,), jnp.int32),
          pltpu.VMEM((CH, D_EMBED), jnp.float32),
      ],
  )
  def k(acts_hbm, table_hbm, out_hbm, tblv, idxv, rowsv):
    cid = lax.axis_index("c")
    sid = lax.axis_index("s")
    wid = sid * NUM_CORES + cid

    pltpu.sync_copy(table_hbm, tblv)
    for i in range(ACT_DIM * D_EMBED // 16):
      sl = pl.ds(i * 16, 16)
      tblv[sl] = _tanh16(tblv[sl])

    base = wid * (n_chunks * CH)
    lane = lax.iota(jnp.int32, 16)

    def chunk_body(g, carry):
      off = base + g * CH
      pltpu.sync_copy(acts_hbm.at[pl.ds(off, CH)], idxv)

      def grp_body(j, c2):
        ivec = idxv[pl.ds(j * 16, 16)]
        src = ivec * D_EMBED
        rvec = lane + j * 16
        cvec = jnp.zeros((16,), jnp.int32)
        for _ in range(D_EMBED):
          vals = plsc.load_gather(tblv, [src])
          plsc.store_scatter(rowsv, [rvec, cvec], vals)
          src = src + 1
          cvec = cvec + 1
        return c2

      lax.fori_loop(0, CH // 16, grp_body, 0)
      pltpu.sync_copy(rowsv, out_hbm.at[pl.ds(off, CH)])
      return carry

    lax.fori_loop(0, n_chunks, chunk_body, 0)

  return k(acts_flat, table_flat)


def kernel(acts, table):
  b, h = acts.shape
  n = b * h
  assert n % (NW * CH) == 0
  n_chunks = n // (NW * CH)
  acts_flat = acts.reshape(n).astype(jnp.int32)
  out = _sc_embed(acts_flat, table.reshape(-1), n_chunks)
  return out.reshape(b, h, D_EMBED)


# speedup vs baseline: 1.8668x; 1.2366x over previous
import functools

import jax
import jax.numpy as jnp
from jax import lax
from jax.experimental import pallas as pl
from jax.experimental.pallas import tpu as pltpu
from jax.experimental.pallas import tpu_sc as plsc

ACT_DIM = 18
D_EMBED = 64

NUM_CORES = 2
NUM_SUBCORES = 16
NW = NUM_CORES * NUM_SUBCORES

CH = 512


def _tanh16(v):
  return 1.0 - 2.0 / (jnp.exp(2.0 * v) + 1.0)


@functools.partial(jax.jit, static_argnames=("n_chunks",))
def _sc_embed(acts_flat, table_flat, n_chunks):
  n = acts_flat.shape[0]

  mesh = plsc.VectorSubcoreMesh(
      core_axis_name="c", subcore_axis_name="s",
      num_cores=NUM_CORES, num_subcores=NUM_SUBCORES)

  @functools.partial(
      pl.kernel,
      out_type=jax.ShapeDtypeStruct((n, D_EMBED), jnp.float32),
      mesh=mesh,
      compiler_params=pltpu.CompilerParams(needs_layout_passes=False),
      scratch_types=[
          pltpu.VMEM((ACT_DIM * D_EMBED,), jnp.float32),
          pltpu.VMEM((CH,), jnp.int32),
          pltpu.VMEM((CH, D_EMBED), jnp.float32),
      ],
  )
  def k(acts_hbm, table_hbm, out_hbm, tblv, idxv, rowsv):
    cid = lax.axis_index("c")
    sid = lax.axis_index("s")
    wid = sid * NUM_CORES + cid

    pltpu.sync_copy(table_hbm, tblv)
    for i in range(ACT_DIM * D_EMBED // 16):
      sl = pl.ds(i * 16, 16)
      tblv[sl] = _tanh16(tblv[sl])

    base = wid * (n_chunks * CH)
    lane = lax.iota(jnp.int32, 16)

    def chunk_body(g, carry):
      off = base + g * CH
      pltpu.sync_copy(acts_hbm.at[pl.ds(off, CH)], idxv)

      @plsc.parallel_loop(0, CH // 16, step=1, unroll=2)
      def grp_body(j):
        ivec = idxv[pl.ds(j * 16, 16)]
        src0 = ivec * D_EMBED
        rvec = lane + j * 16
        for c in range(D_EMBED):
          vals = plsc.load_gather(tblv, [src0 + c])
          cvec = jnp.full((16,), c, jnp.int32)
          plsc.store_scatter(rowsv, [rvec, cvec], vals)
      pltpu.sync_copy(rowsv, out_hbm.at[pl.ds(off, CH)])
      return carry

    lax.fori_loop(0, n_chunks, chunk_body, 0)

  return k(acts_flat, table_flat)


def kernel(acts, table):
  b, h = acts.shape
  n = b * h
  assert n % (NW * CH) == 0
  n_chunks = n // (NW * CH)
  acts_flat = acts.reshape(n).astype(jnp.int32)
  out = _sc_embed(acts_flat, table.reshape(-1), n_chunks)
  return out.reshape(b, h, D_EMBED)


# speedup vs baseline: 1.9456x; 1.0422x over previous
import functools

import jax
import jax.numpy as jnp
from jax import lax
from jax.experimental import pallas as pl
from jax.experimental.pallas import tpu as pltpu
from jax.experimental.pallas import tpu_sc as plsc

ACT_DIM = 18
D_EMBED = 64

NUM_CORES = 2
NUM_SUBCORES = 16
NW = NUM_CORES * NUM_SUBCORES

CH = 512


def _tanh16(v):
  return 1.0 - 2.0 / (jnp.exp(2.0 * v) + 1.0)


@functools.partial(jax.jit, static_argnames=("n_chunks",))
def _sc_embed(acts_flat, table_flat, n_chunks):
  n = acts_flat.shape[0]

  mesh = plsc.VectorSubcoreMesh(
      core_axis_name="c", subcore_axis_name="s",
      num_cores=NUM_CORES, num_subcores=NUM_SUBCORES)

  @functools.partial(
      pl.kernel,
      out_type=jax.ShapeDtypeStruct((n, D_EMBED), jnp.float32),
      mesh=mesh,
      compiler_params=pltpu.CompilerParams(needs_layout_passes=False),
      scratch_types=[
          pltpu.VMEM((ACT_DIM * D_EMBED,), jnp.float32),
          pltpu.VMEM((CH,), jnp.int32),
          pltpu.VMEM((CH, D_EMBED), jnp.float32),
      ],
  )
  def k(acts_hbm, table_hbm, out_hbm, tblv, idxv, rowsv):
    cid = lax.axis_index("c")
    sid = lax.axis_index("s")
    wid = sid * NUM_CORES + cid

    pltpu.sync_copy(table_hbm, tblv)
    for i in range(ACT_DIM * D_EMBED // 16):
      sl = pl.ds(i * 16, 16)
      tblv[sl] = _tanh16(tblv[sl])

    base = wid * (n_chunks * CH)
    lane = lax.iota(jnp.int32, 16)

    def chunk_body(g, carry):
      off = base + g * CH
      pltpu.sync_copy(acts_hbm.at[pl.ds(off, CH)], idxv)

      @plsc.parallel_loop(0, CH // 16, step=1, unroll=2)
      def grp_body(j):
        ivec = idxv[pl.ds(j * 16, 16)]
        src0 = ivec * D_EMBED
        rvec = lane + j * 16
        for cb in range(0, D_EMBED, 16):
          vals = [plsc.load_gather(tblv, [src0 + (cb + t)])
                  for t in range(16)]
          for t in range(16):
            cvec = jnp.full((16,), cb + t, jnp.int32)
            plsc.store_scatter(rowsv, [rvec, cvec], vals[t])
      pltpu.sync_copy(rowsv, out_hbm.at[pl.ds(off, CH)])
      return carry

    lax.fori_loop(0, n_chunks, chunk_body, 0)

  return k(acts_flat, table_flat)


def kernel(acts, table):
  b, h = acts.shape
  n = b * h
  assert n % (NW * CH) == 0
  n_chunks = n // (NW * CH)
  acts_flat = acts.reshape(n).astype(jnp.int32)
  out = _sc_embed(acts_flat, table.reshape(-1), n_chunks)
  return out.reshape(b, h, D_EMBED)


# speedup vs baseline: 4.9660x; 2.5524x over previous
import functools

import jax
import jax.numpy as jnp
from jax import lax
from jax.experimental import pallas as pl
from jax.experimental.pallas import tpu as pltpu
from jax.experimental.pallas import tpu_sc as plsc

ACT_DIM = 18
D_EMBED = 64

NUM_CORES = 2
NUM_SUBCORES = 16
NW = NUM_CORES * NUM_SUBCORES

CH = 512


def _tanh16(v):
  return 1.0 - 2.0 / (jnp.exp(2.0 * v) + 1.0)


@functools.partial(jax.jit, static_argnames=("n_chunks",))
def _sc_embed(acts_flat, table_flat, n_chunks):
  n = acts_flat.shape[0]

  mesh = plsc.VectorSubcoreMesh(
      core_axis_name="c", subcore_axis_name="s",
      num_cores=NUM_CORES, num_subcores=NUM_SUBCORES)

  @functools.partial(
      pl.kernel,
      out_type=jax.ShapeDtypeStruct((n, D_EMBED), jnp.float32),
      mesh=mesh,
      compiler_params=pltpu.CompilerParams(needs_layout_passes=False),
      scratch_types=[
          pltpu.VMEM((ACT_DIM * D_EMBED,), jnp.float32),
          pltpu.VMEM((CH,), jnp.int32),
          pltpu.VMEM((CH, D_EMBED), jnp.float32),
      ],
  )
  def k(acts_hbm, table_hbm, out_hbm, tblv, idxv, rowsv):
    cid = lax.axis_index("c")
    sid = lax.axis_index("s")
    wid = sid * NUM_CORES + cid

    pltpu.sync_copy(table_hbm, tblv)
    for i in range(ACT_DIM * D_EMBED // 16):
      sl = pl.ds(i * 16, 16)
      tblv[sl] = _tanh16(tblv[sl])

    base = wid * (n_chunks * CH)
    lane = lax.iota(jnp.int32, 16)

    def chunk_body(g, carry):
      off = base + g * CH
      pltpu.sync_copy(acts_hbm.at[pl.ds(off, CH)], idxv)

      @plsc.parallel_loop(0, CH // 16, step=1, unroll=2)
      def grp_body(j):
        ivec = idxv[pl.ds(j * 16, 16)]
        src0 = ivec * D_EMBED
        rvec = lane + j * 16
        for cb in range(0, D_EMBED, 16):
          cvecs = [jnp.bitwise_and(lane + (cb + t), D_EMBED - 1)
                   for t in range(16)]
          vals = [plsc.load_gather(tblv, [src0 + cvecs[t]])
                  for t in range(16)]
          for t in range(16):
            plsc.store_scatter(rowsv, [rvec, cvecs[t]], vals[t])
      pltpu.sync_copy(rowsv, out_hbm.at[pl.ds(off, CH)])
      return carry

    lax.fori_loop(0, n_chunks, chunk_body, 0)

  return k(acts_flat, table_flat)


def kernel(acts, table):
  b, h = acts.shape
  n = b * h
  assert n % (NW * CH) == 0
  n_chunks = n // (NW * CH)
  acts_flat = acts.reshape(n).astype(jnp.int32)
  out = _sc_embed(acts_flat, table.reshape(-1), n_chunks)
  return out.reshape(b, h, D_EMBED)
